# baseline (speedup 1.0000x reference)
import functools

import jax
import jax.numpy as jnp
from jax import lax
from jax.experimental import pallas as pl
from jax.experimental.pallas import tpu as pltpu
from jax.experimental.pallas import tpu_sc as plsc

B, S, F = 4, 8192, 2048
NC, NS, L = 2, 16, 16
NW = NC * NS
WPB = NW // B
FPW = F // WPB
R = 64
NCH = S // R


def _sc_cumsum(x):
    mesh = plsc.VectorSubcoreMesh(core_axis_name="c", subcore_axis_name="s")

    @functools.partial(
        pl.kernel,
        mesh=mesh,
        out_type=jax.ShapeDtypeStruct((B, S, F), jnp.float32),
        scratch_types=[
            pltpu.VMEM((R, FPW), jnp.float32),
            pltpu.VMEM((FPW,), jnp.float32),
        ],
    )
    def k(x_hbm, out_hbm, buf, run_ref):
        wid = lax.axis_index("s") * NC + lax.axis_index("c")
        b = wid // WPB
        f0 = (wid % WPB) * FPW

        for c in range(FPW // L):
            run_ref[pl.ds(c * L, L)] = jnp.zeros((L,), jnp.float32)

        def chunk_body(kk, _):
            r0 = kk * R
            pltpu.sync_copy(x_hbm.at[b, pl.ds(r0, R), pl.ds(f0, FPW)], buf)

            def row_body(r, __):
                for c in range(FPW // L):
                    sl = pl.ds(c * L, L)
                    run = run_ref[sl] + buf[r, sl]
                    buf[r, sl] = run
                    run_ref[sl] = run
                return __

            lax.fori_loop(0, R, row_body, 0, unroll=2)
            pltpu.sync_copy(buf, out_hbm.at[b, pl.ds(r0, R), pl.ds(f0, FPW)])
            return _

        lax.fori_loop(0, NCH, chunk_body, 0)

    return k(x)


def kernel(x, dim, dtype):
    return _sc_cumsum(x)


# speedup vs baseline: 4.6896x; 4.6896x over previous
import functools

import jax
import jax.numpy as jnp
from jax import lax
from jax.experimental import pallas as pl
from jax.experimental.pallas import tpu as pltpu
from jax.experimental.pallas import tpu_sc as plsc

B, S, F = 4, 8192, 2048
NC, NS, L = 2, 16, 16
NW = NC * NS
WPB = NW // B
FPW = F // WPB
NLANES = FPW // L
R = 128
NCH = S // R
NP = NCH // 2


def _sc_cumsum(x):
    mesh = plsc.VectorSubcoreMesh(core_axis_name="c", subcore_axis_name="s")

    @functools.partial(
        pl.kernel,
        mesh=mesh,
        out_type=jax.ShapeDtypeStruct((B, S, F), jnp.float32),
        scratch_types=[
            pltpu.VMEM((R, FPW), jnp.float32),
            pltpu.VMEM((R, FPW), jnp.float32),
            pltpu.SemaphoreType.DMA,
            pltpu.SemaphoreType.DMA,
        ],
    )
    def k(x_hbm, out_hbm, buf0, buf1, sem_in, sem_out):
        wid = lax.axis_index("s") * NC + lax.axis_index("c")
        b = wid // WPB
        f0 = (wid % WPB) * FPW

        def src(kk):
            return x_hbm.at[b, pl.ds(kk * R, R), pl.ds(f0, FPW)]

        def dst(kk):
            return out_hbm.at[b, pl.ds(kk * R, R), pl.ds(f0, FPW)]

        def start_in(kk, buf):
            pltpu.make_async_copy(src(kk), buf, sem_in).start()

        def wait_in(buf):
            pltpu.make_async_copy(src(0), buf, sem_in).wait()

        def start_out(kk, buf):
            pltpu.make_async_copy(buf, dst(kk), sem_out).start()

        def wait_out(buf):
            pltpu.make_async_copy(buf, dst(0), sem_out).wait()

        def compute(buf, runs):
            def row_body(r, rs):
                new = []
                for c in range(NLANES):
                    sl = pl.ds(c * L, L)
                    v = rs[c] + buf[r, sl]
                    buf[r, sl] = v
                    new.append(v)
                return tuple(new)

            return lax.fori_loop(0, R, row_body, runs, unroll=2)

        runs0 = tuple(jnp.zeros((L,), jnp.float32) for _ in range(NLANES))
        start_in(0, buf0)

        def pair(p, runs):
            k0 = 2 * p
            @pl.when(p > 0)
            def _():
                wait_out(buf1)

            start_in(k0 + 1, buf1)
            wait_in(buf0)
            runs = compute(buf0, runs)
            start_out(k0, buf0)

            wait_in(buf1)
            runs = compute(buf1, runs)
            start_out(k0 + 1, buf1)

            @pl.when(p + 1 < NP)
            def _():
                wait_out(buf0)
                start_in(k0 + 2, buf0)

            return runs

        lax.fori_loop(0, NP, pair, runs0)
        wait_out(buf0)
        wait_out(buf1)

    return k(x)


def kernel(x, dim, dtype):
    return _sc_cumsum(x)


# speedup vs baseline: 4.7410x; 1.0109x over previous
import functools

import jax
import jax.numpy as jnp
from jax import lax
from jax.experimental import pallas as pl
from jax.experimental.pallas import tpu as pltpu
from jax.experimental.pallas import tpu_sc as plsc

B, S, F = 4, 8192, 2048
NC, NS, L = 2, 16, 16
NW = NC * NS
WPB = NW // B
FPW = F // WPB
NLANES = FPW // L
R = 64
NCH = S // R
NBUF = 4
PD = NBUF - 1
NT = NCH // NBUF
assert NCH % NBUF == 0


def _sc_cumsum(x):
    mesh = plsc.VectorSubcoreMesh(core_axis_name="c", subcore_axis_name="s")

    @functools.partial(
        pl.kernel,
        mesh=mesh,
        out_type=jax.ShapeDtypeStruct((B, S, F), jnp.float32),
        scratch_types=[
            pltpu.VMEM((R, FPW), jnp.float32),
            pltpu.VMEM((R, FPW), jnp.float32),
            pltpu.VMEM((R, FPW), jnp.float32),
            pltpu.VMEM((R, FPW), jnp.float32),
            pltpu.SemaphoreType.DMA,
            pltpu.SemaphoreType.DMA,
        ],
    )
    def k(x_hbm, out_hbm, buf0, buf1, buf2, buf3, sem_in, sem_out):
        bufs = (buf0, buf1, buf2, buf3)
        wid = lax.axis_index("s") * NC + lax.axis_index("c")
        b = wid // WPB
        f0 = (wid % WPB) * FPW

        def src(kk):
            return x_hbm.at[b, pl.ds(kk * R, R), pl.ds(f0, FPW)]

        def dst(kk):
            return out_hbm.at[b, pl.ds(kk * R, R), pl.ds(f0, FPW)]

        def start_in(kk, buf):
            pltpu.make_async_copy(src(kk), buf, sem_in).start()

        def wait_in(buf):
            pltpu.make_async_copy(src(0), buf, sem_in).wait()

        def start_out(kk, buf):
            pltpu.make_async_copy(buf, dst(kk), sem_out).start()

        def wait_out(buf):
            pltpu.make_async_copy(buf, dst(0), sem_out).wait()

        def compute(buf, runs):
            def row_body(r, rs):
                new = []
                for c in range(NLANES):
                    sl = pl.ds(c * L, L)
                    v = rs[c] + buf[r, sl]
                    buf[r, sl] = v
                    new.append(v)
                return tuple(new)

            return lax.fori_loop(0, R, row_body, runs, unroll=2)

        runs0 = tuple(jnp.zeros((L,), jnp.float32) for _ in range(NLANES))
        for j in range(PD):
            start_in(j, bufs[j])

        def ring(t, runs):
            k0 = NBUF * t
            for j in range(NBUF):
                kk = k0 + j
                buf = bufs[j]
                wait_in(buf)
                runs = compute(buf, runs)
                start_out(kk, buf)
                nxt = bufs[(j + PD) % NBUF]

                @pl.when(kk + PD < NCH)
                def _():
                    @pl.when(kk >= 1)
                    def _():
                        wait_out(nxt)

                    start_in(kk + PD, nxt)

            return runs

        lax.fori_loop(0, NT, ring, runs0)
        for j in range(NBUF):
            wait_out(bufs[(j + 1) % NBUF])

    return k(x)


def kernel(x, dim, dtype):
    return _sc_cumsum(x)
